# initial kernel scaffold (unmeasured)
import jax
import jax.numpy as jnp
from jax import lax
from jax.experimental import pallas as pl
from jax.experimental.pallas import tpu as pltpu

N_DEV = 4
SQ = 2048
SKV = 2048
D_MODEL = 1024
H_PER = 8
DH = 128
SCALE = 0.08838834764831843
QB = 256
BAND = 768


def _body(x_ref, w_ref, k_hbm, v_hbm, out_ref, comm, q_ref, ctx_ref,
          k_vm, v_vm, send_sems, recv_sems, dma_sems):
    my = lax.axis_index("i")
    right = lax.rem(my + 1, N_DEV)
    left = lax.rem(my + 3, N_DEV)

    barrier = pltpu.get_barrier_semaphore()
    for nbr in (left, right):
        pl.semaphore_signal(barrier, inc=1, device_id=(nbr,),
                            device_id_type=pl.DeviceIdType.MESH)
    pl.semaphore_wait(barrier, 2)

    def wq_of(r):
        if r == 0:
            return w_ref[0:D_MODEL, :]
        return comm[r - 1, 0:D_MODEL, :]

    def wo_of(r):
        if r == 0:
            return w_ref[D_MODEL:2 * D_MODEL, :]
        return comm[r - 1, D_MODEL:2 * D_MODEL, :]

    def compute_block(r):
        slot = r % 2
        origin = lax.rem(my - r + N_DEV, N_DEV)
        hd0 = origin * H_PER
        ck = pltpu.make_async_copy(
            k_hbm.at[pl.ds(hd0, H_PER)], k_vm.at[slot], dma_sems.at[slot, 0])
        cv = pltpu.make_async_copy(
            v_hbm.at[pl.ds(hd0, H_PER)], v_vm.at[slot], dma_sems.at[slot, 1])
        ck.start()
        cv.start()

        q_ref[...] = (
            jnp.dot(x_ref[...], wq_of(r), preferred_element_type=jnp.float32)
            * SCALE
        ).astype(jnp.bfloat16)
        ck.wait()
        cv.wait()

        def head(h, carry):
            def do_qb(qb_start, pieces):
                qh = q_ref[pl.ds(qb_start, QB), pl.ds(h * DH, DH)]
                ss = []
                for lo, width in pieces:
                    kp = k_vm[slot, h, pl.ds(lo, width), :]
                    s = lax.dot_general(
                        qh, kp, (((1,), (1,)), ((), ())),
                        preferred_element_type=jnp.float32)
                    qi = qb_start + lax.broadcasted_iota(
                        jnp.int32, (QB, width), 0)
                    ki = lo + lax.broadcasted_iota(jnp.int32, (QB, width), 1)
                    mask = (jnp.abs(qi - ki) <= 128) | (ki < 32) | (qi < 32)
                    ss.append(jnp.where(mask, s, -1e9))
                m = ss[0].max(axis=1, keepdims=True)
                for s in ss[1:]:
                    m = jnp.maximum(m, s.max(axis=1, keepdims=True))
                es = [jnp.exp(s - m) for s in ss]
                denom = es[0].sum(axis=1, keepdims=True)
                for e in es[1:]:
                    denom = denom + e.sum(axis=1, keepdims=True)
                acc = None
                for e, (lo, width) in zip(es, pieces):
                    vp = v_vm[slot, h, pl.ds(lo, width), :]
                    pv = jnp.dot(e.astype(jnp.bfloat16), vp,
                                 preferred_element_type=jnp.float32)
                    acc = pv if acc is None else acc + pv
                ctx_ref[pl.ds(qb_start, QB), pl.ds(h * DH, DH)] = (
                    acc / denom).astype(jnp.bfloat16)

            do_qb(0, [(0, SKV)])
            do_qb(QB, [(0, 3 * QB)])

            def qb_loop(qb, c):
                lo = jnp.minimum((qb - 1) * QB, SKV - BAND)
                do_qb(qb * QB, [(0, QB), (lo, BAND)])
                return c
            lax.fori_loop(2, SQ // QB, qb_loop, 0)
            return carry

        lax.fori_loop(0, H_PER, head, 0)

        part = jnp.dot(ctx_ref[...], wo_of(r),
                       preferred_element_type=jnp.float32)
        if r == 0:
            out_ref[0] = part
        else:
            out_ref[0] = out_ref[0] + part

    for hp in range(N_DEV - 1):
        src = w_ref if hp == 0 else comm.at[hp - 1]
        rdma = pltpu.make_async_remote_copy(
            src_ref=src,
            dst_ref=comm.at[hp],
            send_sem=send_sems.at[hp],
            recv_sem=recv_sems.at[hp],
            device_id=(right,),
            device_id_type=pl.DeviceIdType.MESH,
        )
        rdma.start()
        compute_block(hp)
        rdma.wait()
    compute_block(N_DEV - 1)


def kernel(x, Wq, K_ext, V_ext, Wo):
    my = lax.axis_index("i")
    xb = x[0].astype(jnp.bfloat16)
    w_my = jnp.concatenate(
        [Wq.astype(jnp.bfloat16), Wo.astype(jnp.bfloat16)], axis=0
    )
    kb = jnp.transpose(
        lax.dynamic_index_in_dim(K_ext, my, 0, keepdims=False), (1, 0, 2)
    ).astype(jnp.bfloat16)
    vb = jnp.transpose(
        lax.dynamic_index_in_dim(V_ext, my, 0, keepdims=False), (1, 0, 2)
    ).astype(jnp.bfloat16)

    return pl.pallas_call(
        _body,
        out_shape=jax.ShapeDtypeStruct((1, SQ, D_MODEL), jnp.float32),
        in_specs=[
            pl.BlockSpec(memory_space=pltpu.VMEM),
            pl.BlockSpec(memory_space=pltpu.VMEM),
            pl.BlockSpec(memory_space=pltpu.ANY),
            pl.BlockSpec(memory_space=pltpu.ANY),
        ],
        out_specs=pl.BlockSpec(memory_space=pltpu.VMEM),
        scratch_shapes=[
            pltpu.VMEM((N_DEV - 1, 2 * D_MODEL, D_MODEL), jnp.bfloat16),
            pltpu.VMEM((SQ, D_MODEL), jnp.bfloat16),
            pltpu.VMEM((SQ, D_MODEL), jnp.bfloat16),
            pltpu.VMEM((2, H_PER, SKV, DH), jnp.bfloat16),
            pltpu.VMEM((2, H_PER, SKV, DH), jnp.bfloat16),
            pltpu.SemaphoreType.DMA((N_DEV - 1,)),
            pltpu.SemaphoreType.DMA((N_DEV - 1,)),
            pltpu.SemaphoreType.DMA((2, 2)),
        ],
        compiler_params=pltpu.CompilerParams(collective_id=0),
    )(xb, w_my, kb, vb)


# baseline (device time: 367351 ns/iter reference)
import jax
import jax.numpy as jnp
from jax import lax
from jax.experimental import pallas as pl
from jax.experimental.pallas import tpu as pltpu

N_DEV = 4
SQ = 2048
SKV = 2048
D_MODEL = 1024
H_PER = 8
DH = 128
SCALE = 0.08838834764831843
QB = 256
BAND = 768


def _body(x_ref, w_ref, k_hbm, v_hbm, out_ref, comm, q_ref, ctx_ref,
          k_vm, v_vm, send_sems, recv_sems, dma_sems):
    my = lax.axis_index("i")
    right = lax.rem(my + 1, N_DEV)
    left = lax.rem(my + 3, N_DEV)

    barrier = pltpu.get_barrier_semaphore()
    for nbr in (left, right):
        pl.semaphore_signal(barrier, inc=1, device_id=(nbr,),
                            device_id_type=pl.DeviceIdType.MESH)
    pl.semaphore_wait(barrier, 2)

    def wq_of(r):
        if r == 0:
            return w_ref[0:D_MODEL, :]
        return comm[r - 1, 0:D_MODEL, :]

    def wo_of(r):
        if r == 0:
            return w_ref[D_MODEL:2 * D_MODEL, :]
        return comm[r - 1, D_MODEL:2 * D_MODEL, :]

    def compute_block(r):
        slot = r % 2
        origin = lax.rem(my - r + N_DEV, N_DEV)
        hd0 = origin * H_PER
        ck = pltpu.make_async_copy(
            k_hbm.at[pl.ds(hd0, H_PER)], k_vm.at[slot], dma_sems.at[slot, 0])
        cv = pltpu.make_async_copy(
            v_hbm.at[pl.ds(hd0, H_PER)], v_vm.at[slot], dma_sems.at[slot, 1])
        ck.start()
        cv.start()

        q_ref[...] = (
            jnp.dot(x_ref[...], wq_of(r), preferred_element_type=jnp.float32)
            * SCALE
        ).astype(jnp.bfloat16)
        ck.wait()
        cv.wait()

        def head(h, carry):
            def do_qb(qb_start, pieces):
                qh = q_ref[pl.ds(qb_start, QB), pl.ds(h * DH, DH)]
                ss = []
                for lo, width in pieces:
                    kp = k_vm[slot, h, pl.ds(lo, width), :]
                    s = lax.dot_general(
                        qh, kp, (((1,), (1,)), ((), ())),
                        preferred_element_type=jnp.float32)
                    qi = qb_start + lax.broadcasted_iota(
                        jnp.int32, (QB, width), 0)
                    ki = lo + lax.broadcasted_iota(jnp.int32, (QB, width), 1)
                    mask = (jnp.abs(qi - ki) <= 128) | (ki < 32) | (qi < 32)
                    ss.append(jnp.where(mask, s, -1e9))
                m = ss[0].max(axis=1, keepdims=True)
                for s in ss[1:]:
                    m = jnp.maximum(m, s.max(axis=1, keepdims=True))
                es = [jnp.exp(s - m) for s in ss]
                denom = es[0].sum(axis=1, keepdims=True)
                for e in es[1:]:
                    denom = denom + e.sum(axis=1, keepdims=True)
                acc = None
                for e, (lo, width) in zip(es, pieces):
                    vp = v_vm[slot, h, pl.ds(lo, width), :]
                    pv = jnp.dot(e.astype(jnp.bfloat16), vp,
                                 preferred_element_type=jnp.float32)
                    acc = pv if acc is None else acc + pv
                ctx_ref[pl.ds(qb_start, QB), pl.ds(h * DH, DH)] = (
                    acc / denom).astype(jnp.bfloat16)

            do_qb(0, [(0, SKV)])
            do_qb(QB, [(0, 3 * QB)])

            def qb_loop(qb, c):
                lo = jnp.minimum((qb - 1) * QB, SKV - BAND)
                do_qb(qb * QB, [(0, QB), (lo, BAND)])
                return c
            lax.fori_loop(2, SQ // QB, qb_loop, 0)
            return carry

        lax.fori_loop(0, H_PER, head, 0)

        part = jnp.dot(ctx_ref[...], wo_of(r),
                       preferred_element_type=jnp.float32)
        if r == 0:
            out_ref[0] = part
        else:
            out_ref[0] = out_ref[0] + part

    for hp in range(N_DEV - 1):
        src = w_ref if hp == 0 else comm.at[hp - 1]
        rdma = pltpu.make_async_remote_copy(
            src_ref=src,
            dst_ref=comm.at[hp],
            send_sem=send_sems.at[hp],
            recv_sem=recv_sems.at[hp],
            device_id=(right,),
            device_id_type=pl.DeviceIdType.MESH,
        )
        rdma.start()
        compute_block(hp)
        rdma.wait()
    compute_block(N_DEV - 1)


def kernel(x, Wq, K_ext, V_ext, Wo):
    my = lax.axis_index("i")
    xb = x[0].astype(jnp.bfloat16)
    w_my = jnp.concatenate(
        [Wq.astype(jnp.bfloat16), Wo.astype(jnp.bfloat16)], axis=0
    )
    kb = jnp.transpose(
        lax.dynamic_index_in_dim(K_ext, my, 0, keepdims=False), (1, 0, 2)
    ).astype(jnp.bfloat16)
    vb = jnp.transpose(
        lax.dynamic_index_in_dim(V_ext, my, 0, keepdims=False), (1, 0, 2)
    ).astype(jnp.bfloat16)

    return pl.pallas_call(
        _body,
        out_shape=jax.ShapeDtypeStruct((1, SQ, D_MODEL), jnp.float32),
        in_specs=[
            pl.BlockSpec(memory_space=pltpu.VMEM),
            pl.BlockSpec(memory_space=pltpu.VMEM),
            pl.BlockSpec(memory_space=pl.ANY),
            pl.BlockSpec(memory_space=pl.ANY),
        ],
        out_specs=pl.BlockSpec(memory_space=pltpu.VMEM),
        scratch_shapes=[
            pltpu.VMEM((N_DEV - 1, 2 * D_MODEL, D_MODEL), jnp.bfloat16),
            pltpu.VMEM((SQ, D_MODEL), jnp.bfloat16),
            pltpu.VMEM((SQ, D_MODEL), jnp.bfloat16),
            pltpu.VMEM((2, H_PER, SKV, DH), jnp.bfloat16),
            pltpu.VMEM((2, H_PER, SKV, DH), jnp.bfloat16),
            pltpu.SemaphoreType.DMA((N_DEV - 1,)),
            pltpu.SemaphoreType.DMA((N_DEV - 1,)),
            pltpu.SemaphoreType.DMA((2, 2)),
        ],
        compiler_params=pltpu.CompilerParams(
            collective_id=0, vmem_limit_bytes=100 * 1024 * 1024),
    )(xb, w_my, kb, vb)
